# baseline (device time: 196675 ns/iter reference)
import jax
import jax.numpy as jnp
from jax import lax
from jax.experimental import pallas as pl
from jax.experimental.pallas import tpu as pltpu

B, S, HD_IN, HD_OUT = 4, 1024, 2048, 4096
S_HALF = S // 2
C = HD_OUT // 2
NC = 8
R = B * S_HALF // NC


def kernel(O, Wo):
    H, D = O.shape[2], O.shape[3]
    Wo2 = Wo.astype(jnp.bfloat16)

    def body(
        o_hbm, wo_ref, out_hbm,
        land_rem, land_own, xsend, xrecv, yrecv, own, stage,
        xsend_sems, xrecv_sems, fsend_sems, yrecv_sems,
        rem_sem, own_sem, store_sem,
    ):
        my_x = lax.axis_index("x")
        my_y = lax.axis_index("y")
        x_nbr = (1 - my_x, my_y)
        y_nbr = (my_x, 1 - my_y)

        barrier_sem = pltpu.get_barrier_semaphore()
        for nbr in (x_nbr, y_nbr):
            pl.semaphore_signal(
                barrier_sem, inc=1,
                device_id=nbr, device_id_type=pl.DeviceIdType.MESH,
            )
        pl.semaphore_wait(barrier_sem, 2)

        own_rows = my_x * S_HALF
        rem_rows = (1 - my_x) * S_HALF

        def xrdma(c):
            return pltpu.make_async_remote_copy(
                src_ref=xsend.at[c % 2],
                dst_ref=xrecv.at[c],
                send_sem=xsend_sems.at[c % 2],
                recv_sem=xrecv_sems.at[c],
                device_id=x_nbr,
                device_id_type=pl.DeviceIdType.MESH,
            )

        def fwd(c):
            return pltpu.make_async_remote_copy(
                src_ref=xrecv.at[c],
                dst_ref=yrecv.at[c],
                send_sem=fsend_sems.at[c],
                recv_sem=yrecv_sems.at[c],
                device_id=y_nbr,
                device_id_type=pl.DeviceIdType.MESH,
            )

        def store(c):
            b, j = divmod(c, NC // B)
            return pltpu.make_async_copy(
                stage, out_hbm.at[b, pl.ds(j * R, R), :], store_sem
            )

        def consume(c):
            fwd(c).wait_recv()
            if c > 0:
                store(c - 1).wait()

            @pl.when(my_y == 0)
            def _():
                stage[:, :C] = (
                    own[c % 2, :, :C].astype(jnp.float32)
                    + xrecv[c].astype(jnp.float32)
                ).astype(jnp.bfloat16)
                stage[:, C:] = (
                    own[c % 2, :, C:].astype(jnp.float32)
                    + yrecv[c].astype(jnp.float32)
                ).astype(jnp.bfloat16)

            @pl.when(my_y == 1)
            def _():
                stage[:, :C] = (
                    own[c % 2, :, :C].astype(jnp.float32)
                    + yrecv[c].astype(jnp.float32)
                ).astype(jnp.bfloat16)
                stage[:, C:] = (
                    own[c % 2, :, C:].astype(jnp.float32)
                    + xrecv[c].astype(jnp.float32)
                ).astype(jnp.bfloat16)

            store(c).start()

        def head_gather(c, half_rows, land, sem):
            b, j = divmod(c, NC // B)
            copies = [
                pltpu.make_async_copy(
                    o_hbm.at[b, pl.ds(half_rows + j * R, R), h, :],
                    land.at[:, pl.ds(h * D, D)],
                    sem,
                )
                for h in range(H)
            ]
            for cp in copies:
                cp.start()
            return copies

        for c in range(NC):
            rem_copies = head_gather(c, rem_rows, land_rem, rem_sem)
            own_copies = head_gather(c, own_rows, land_own, own_sem)

            if c >= 2:
                consume(c - 2)

            if c >= 2:
                xrdma(c - 2).wait_send()

            for cp in rem_copies:
                cp.wait()
            o_rem = land_rem[...].astype(jnp.bfloat16)

            @pl.when(my_y == 0)
            def _():
                xsend[c % 2] = jnp.dot(
                    o_rem, wo_ref[:, :C],
                    preferred_element_type=jnp.float32,
                ).astype(jnp.bfloat16)

            @pl.when(my_y == 1)
            def _():
                xsend[c % 2] = jnp.dot(
                    o_rem, wo_ref[:, C:],
                    preferred_element_type=jnp.float32,
                ).astype(jnp.bfloat16)

            xrdma(c).start()

            for cp in own_copies:
                cp.wait()
            o_own = land_own[...].astype(jnp.bfloat16)
            own[c % 2, :, :C] = jnp.dot(
                o_own, wo_ref[:, :C], preferred_element_type=jnp.float32
            ).astype(jnp.bfloat16)
            own[c % 2, :, C:] = jnp.dot(
                o_own, wo_ref[:, C:], preferred_element_type=jnp.float32
            ).astype(jnp.bfloat16)

            xrdma(c).wait_recv()
            fwd(c).start()

        consume(NC - 2)
        consume(NC - 1)
        for c in range(NC - 2, NC):
            xrdma(c).wait_send()
        for c in range(NC):
            fwd(c).wait_send()
        store(NC - 1).wait()

    return pl.pallas_call(
        body,
        out_shape=jax.ShapeDtypeStruct((B, S_HALF, HD_OUT), jnp.bfloat16),
        in_specs=[
            pl.BlockSpec(memory_space=pl.ANY),
            pl.BlockSpec(memory_space=pltpu.VMEM),
        ],
        out_specs=pl.BlockSpec(memory_space=pl.ANY),
        scratch_shapes=[
            pltpu.VMEM((R, HD_IN), jnp.float32),
            pltpu.VMEM((R, HD_IN), jnp.float32),
            pltpu.VMEM((2, R, C), jnp.bfloat16),
            pltpu.VMEM((NC, R, C), jnp.bfloat16),
            pltpu.VMEM((NC, R, C), jnp.bfloat16),
            pltpu.VMEM((2, R, HD_OUT), jnp.bfloat16),
            pltpu.VMEM((R, HD_OUT), jnp.bfloat16),
            pltpu.SemaphoreType.DMA((2,)),
            pltpu.SemaphoreType.DMA((NC,)),
            pltpu.SemaphoreType.DMA((NC,)),
            pltpu.SemaphoreType.DMA((NC,)),
            pltpu.SemaphoreType.DMA,
            pltpu.SemaphoreType.DMA,
            pltpu.SemaphoreType.DMA,
        ],
        compiler_params=pltpu.CompilerParams(
            collective_id=0,
            vmem_limit_bytes=64 * 1024 * 1024,
        ),
    )(O, Wo2)
